# Initial kernel scaffold; baseline (speedup 1.0000x reference)
import jax
import jax.numpy as jnp
from jax.experimental import pallas as pl


def kernel(x, edge_index, edge_weights, W_w, b_w, att):
    raise NotImplementedError("write your pallas kernel here")


# speedup vs baseline: 59.9309x; 59.9309x over previous
import functools

import jax
import jax.numpy as jnp
from jax import lax
from jax.experimental import pallas as pl
from jax.experimental.pallas import tpu as pltpu
from jax.experimental.pallas import tpu_sc as plsc

NC = 2
NS = 16
LANES = 16
CHUNK = 128


def _sc_scatter_partials(x, src_p, dst_p, n, d, chunks_per_worker):
    rows_per_tile = -(-n // (NS * 8)) * 8
    n_pad = rows_per_tile * NS

    mesh = plsc.VectorSubcoreMesh(core_axis_name="c", subcore_axis_name="s")

    @functools.partial(
        pl.kernel,
        out_type=jax.ShapeDtypeStruct((NC * n_pad, d), jnp.float32),
        mesh=mesh,
        scratch_types=[
            pltpu.VMEM((CHUNK,), jnp.int32),
            pltpu.VMEM((CHUNK,), jnp.int32),
            pltpu.VMEM((CHUNK, d), jnp.float32),
            pltpu.VMEM_SHARED((n_pad, d), jnp.float32),
            pltpu.SemaphoreType.DMA,
        ],
    )
    def scatter_kernel(x_hbm, src_hbm, dst_hbm, out_hbm, sidx, didx, rows, acc, sem):
        cid = lax.axis_index("c")
        sid = lax.axis_index("s")
        wid = cid * NS + sid

        def zero_body(t, _):
            rows[t // (d // LANES), pl.ds((t % (d // LANES)) * LANES, LANES)] = (
                jnp.zeros((LANES,), jnp.float32))
            return 0
        lax.fori_loop(0, CHUNK * (d // LANES), zero_body, 0)
        r0 = sid * rows_per_tile
        full = rows_per_tile // CHUNK
        for k in range(full):
            pltpu.sync_copy(rows, acc.at[pl.ds(r0 + k * CHUNK, CHUNK)])
        rem = rows_per_tile - full * CHUNK
        if rem:
            pltpu.sync_copy(rows.at[pl.ds(0, rem)],
                            acc.at[pl.ds(r0 + full * CHUNK, rem)])
        plsc.subcore_barrier()

        def edge_body(i, _):
            base = (wid * chunks_per_worker + i) * CHUNK
            pltpu.sync_copy(src_hbm.at[pl.ds(base, CHUNK)], sidx)
            pltpu.sync_copy(dst_hbm.at[pl.ds(base, CHUNK)], didx)
            pltpu.async_copy(x_hbm.at[sidx], rows, sem).wait()
            pltpu.sync_copy(rows, acc.at[didx], add=True)
            return 0
        lax.fori_loop(0, chunks_per_worker, edge_body, 0)
        plsc.subcore_barrier()

        pltpu.sync_copy(acc.at[pl.ds(r0, rows_per_tile)],
                        out_hbm.at[pl.ds(cid * n_pad + r0, rows_per_tile)])

    return scatter_kernel(x, src_p, dst_p), n_pad


def _combine(x, p0, p1, n, d, scale):
    block = 2000

    def body(x_ref, a_ref, b_ref, o_ref):
        o_ref[...] = jnp.maximum(
            (x_ref[...] + a_ref[...] + b_ref[...]) * scale, 0.0)

    spec = pl.BlockSpec((block, d), lambda i: (i, 0))
    return pl.pallas_call(
        body,
        grid=(n // block,),
        in_specs=[spec, spec, spec],
        out_specs=spec,
        out_shape=jax.ShapeDtypeStruct((n, d), jnp.float32),
    )(x, p0, p1)


def kernel(x, edge_index, edge_weights, W_w, b_w, att):
    n, d = x.shape
    e = edge_index.shape[1]
    out_dim = att.shape[1]

    per_worker_edges = -(-e // (NC * NS * CHUNK)) * CHUNK
    ep = per_worker_edges * NC * NS
    pad = ep - e
    src_p = jnp.concatenate([edge_index[0], jnp.zeros((pad,), jnp.int32)])
    dst_p = jnp.concatenate([edge_index[1], jnp.full((pad,), n, jnp.int32)])

    partials, n_pad = _sc_scatter_partials(x, src_p, dst_p, n, d,
                                           per_worker_edges // CHUNK)
    return _combine(x, partials[:n], partials[n_pad:n_pad + n], n, d,
                    1.0 / out_dim)


# speedup vs baseline: 72.7131x; 1.2133x over previous
import functools

import jax
import jax.numpy as jnp
from jax import lax
from jax.experimental import pallas as pl
from jax.experimental.pallas import tpu as pltpu
from jax.experimental.pallas import tpu_sc as plsc

NC = 2
NS = 16
LANES = 16
CHUNK = 128


def _sc_scatter_partials(x, src_p, dst_p, n, d, chunks_per_worker):
    rows_per_tile = -(-n // (NS * 8)) * 8
    n_pad = rows_per_tile * NS

    mesh = plsc.VectorSubcoreMesh(core_axis_name="c", subcore_axis_name="s")

    @functools.partial(
        pl.kernel,
        out_type=jax.ShapeDtypeStruct((NC * n_pad, d), jnp.float32),
        mesh=mesh,
        scratch_types=[
            pltpu.VMEM((chunks_per_worker, CHUNK), jnp.int32),
            pltpu.VMEM((chunks_per_worker, CHUNK), jnp.int32),
            pltpu.VMEM((CHUNK, d), jnp.float32),
            pltpu.VMEM((CHUNK, d), jnp.float32),
            pltpu.VMEM_SHARED((n_pad, d), jnp.float32),
            pltpu.SemaphoreType.DMA,
            pltpu.SemaphoreType.DMA,
        ],
    )
    def scatter_kernel(x_hbm, src_hbm, dst_hbm, out_hbm,
                       sidx, didx, rows0, rows1, acc, sem0, sem1):
        cid = lax.axis_index("c")
        sid = lax.axis_index("s")
        wid = cid * NS + sid

        def zero_body(t, _):
            rows0[t // (d // LANES), pl.ds((t % (d // LANES)) * LANES, LANES)] = (
                jnp.zeros((LANES,), jnp.float32))
            return 0
        lax.fori_loop(0, CHUNK * (d // LANES), zero_body, 0)
        r0 = sid * rows_per_tile
        full = rows_per_tile // CHUNK
        for k in range(full):
            pltpu.sync_copy(rows0, acc.at[pl.ds(r0 + k * CHUNK, CHUNK)])
        rem = rows_per_tile - full * CHUNK
        if rem:
            pltpu.sync_copy(rows0.at[pl.ds(0, rem)],
                            acc.at[pl.ds(r0 + full * CHUNK, rem)])
        plsc.subcore_barrier()

        cbase = wid * chunks_per_worker
        pltpu.sync_copy(src_hbm.at[pl.ds(cbase, chunks_per_worker)], sidx)
        pltpu.sync_copy(dst_hbm.at[pl.ds(cbase, chunks_per_worker)], didx)

        half = chunks_per_worker // 2
        pltpu.async_copy(x_hbm.at[sidx.at[0]], rows0, sem0)

        def pipe_body(jj, _):
            c0 = 2 * jj
            pltpu.async_copy(x_hbm.at[sidx.at[c0 + 1]], rows1, sem1)
            pltpu.make_async_copy(x_hbm.at[sidx.at[c0]], rows0, sem0).wait()
            pltpu.sync_copy(rows0, acc.at[didx.at[c0]], add=True)

            @pl.when(jj + 1 < half)
            def _prefetch():
                pltpu.async_copy(x_hbm.at[sidx.at[c0 + 2]], rows0, sem0)

            pltpu.make_async_copy(x_hbm.at[sidx.at[c0 + 1]], rows1, sem1).wait()
            pltpu.sync_copy(rows1, acc.at[didx.at[c0 + 1]], add=True)
            return 0
        lax.fori_loop(0, half, pipe_body, 0)
        plsc.subcore_barrier()

        pltpu.sync_copy(acc.at[pl.ds(r0, rows_per_tile)],
                        out_hbm.at[pl.ds(cid * n_pad + r0, rows_per_tile)])

    return scatter_kernel(x, src_p, dst_p), n_pad


def _combine(x, p0, p1, n, d, scale):
    block = 2000

    def body(x_ref, a_ref, b_ref, o_ref):
        o_ref[...] = jnp.maximum(
            (x_ref[...] + a_ref[...] + b_ref[...]) * scale, 0.0)

    spec = pl.BlockSpec((block, d), lambda i: (i, 0))
    return pl.pallas_call(
        body,
        grid=(n // block,),
        in_specs=[spec, spec, spec],
        out_specs=spec,
        out_shape=jax.ShapeDtypeStruct((n, d), jnp.float32),
    )(x, p0, p1)


def kernel(x, edge_index, edge_weights, W_w, b_w, att):
    n, d = x.shape
    e = edge_index.shape[1]
    out_dim = att.shape[1]

    cpw = -(-e // (NC * NS * CHUNK))
    cpw += cpw % 2
    per_worker_edges = cpw * CHUNK
    ep = per_worker_edges * NC * NS
    pad = ep - e
    src_p = jnp.concatenate(
        [edge_index[0], jnp.zeros((pad,), jnp.int32)]).reshape(-1, CHUNK)
    dst_p = jnp.concatenate(
        [edge_index[1], jnp.full((pad,), n, jnp.int32)]).reshape(-1, CHUNK)

    partials, n_pad = _sc_scatter_partials(x, src_p, dst_p, n, d,
                                           per_worker_edges // CHUNK)
    return _combine(x, partials[:n], partials[n_pad:n_pad + n], n, d,
                    1.0 / out_dim)


# speedup vs baseline: 75.8619x; 1.0433x over previous
import functools

import jax
import jax.numpy as jnp
from jax import lax
from jax.experimental import pallas as pl
from jax.experimental.pallas import tpu as pltpu
from jax.experimental.pallas import tpu_sc as plsc

NC = 2
NS = 16
LANES = 16
CHUNK = 128
NBUF = 2
FRAC0 = 0.775


def _split_chunks(e):
    g = 8 * NBUF // __import__("math").gcd(8, NBUF)
    tot = -(-e // (NS * CHUNK))
    tot = -(-tot // g) * g
    cpw0 = int(round(tot * FRAC0 / g)) * g
    cpw0 = max(g, min(cpw0, tot - g))
    return cpw0, tot - cpw0


def _sc_scatter_partials(x, src_p, dst_p, n, d, cpw0, cpw1):
    rows_per_tile = -(-n // (NS * 8)) * 8
    n_pad = rows_per_tile * NS
    cpw_max = max(cpw0, cpw1)

    mesh = plsc.VectorSubcoreMesh(core_axis_name="c", subcore_axis_name="s")

    @functools.partial(
        pl.kernel,
        out_type=jax.ShapeDtypeStruct((NC * n_pad, d), jnp.float32),
        mesh=mesh,
        scratch_types=[
            pltpu.VMEM((cpw_max, CHUNK), jnp.int32),
            pltpu.VMEM((cpw_max, CHUNK), jnp.int32),
            *[pltpu.VMEM((CHUNK, d), jnp.float32) for _ in range(NBUF)],
            pltpu.VMEM_SHARED((n_pad, d), jnp.float32),
            *[pltpu.SemaphoreType.DMA for _ in range(NBUF)],
        ],
    )
    def scatter_kernel(x_hbm, src_hbm, dst_hbm, out_hbm,
                       sidx, didx, *rest):
        rows = rest[:NBUF]
        acc = rest[NBUF]
        sems = rest[NBUF + 1:]
        cid = lax.axis_index("c")
        sid = lax.axis_index("s")

        zbuf = rows[0]
        def zero_body(t, _):
            zbuf[t // (d // LANES), pl.ds((t % (d // LANES)) * LANES, LANES)] = (
                jnp.zeros((LANES,), jnp.float32))
            return 0
        lax.fori_loop(0, CHUNK * (d // LANES), zero_body, 0)
        r0 = sid * rows_per_tile
        full = rows_per_tile // CHUNK
        for k in range(full):
            pltpu.sync_copy(zbuf, acc.at[pl.ds(r0 + k * CHUNK, CHUNK)])
        rem = rows_per_tile - full * CHUNK
        if rem:
            pltpu.sync_copy(zbuf.at[pl.ds(0, rem)],
                            acc.at[pl.ds(r0 + full * CHUNK, rem)])
        plsc.subcore_barrier()

        cpw = jnp.where(cid == 0, cpw0, cpw1)
        w = cid * NS + sid

        @pl.when(cid == 0)
        def _stage0():
            pltpu.sync_copy(src_hbm.at[w, pl.ds(0, cpw0)],
                            sidx.at[pl.ds(0, cpw0)])
            pltpu.sync_copy(dst_hbm.at[w, pl.ds(0, cpw0)],
                            didx.at[pl.ds(0, cpw0)])

        @pl.when(cid == 1)
        def _stage1():
            pltpu.sync_copy(src_hbm.at[w, pl.ds(0, cpw1)],
                            sidx.at[pl.ds(0, cpw1)])
            pltpu.sync_copy(dst_hbm.at[w, pl.ds(0, cpw1)],
                            didx.at[pl.ds(0, cpw1)])

        for b in range(NBUF):
            pltpu.async_copy(x_hbm.at[sidx.at[b]], rows[b], sems[b])

        def pipe_body(jj, _):
            for b in range(NBUF):
                c = NBUF * jj + b
                pltpu.make_async_copy(
                    x_hbm.at[sidx.at[c]], rows[b], sems[b]).wait()
                pltpu.sync_copy(rows[b], acc.at[didx.at[c]], add=True)

                @pl.when(c + NBUF < cpw)
                def _prefetch():
                    pltpu.async_copy(
                        x_hbm.at[sidx.at[c + NBUF]], rows[b], sems[b])
            return 0
        lax.fori_loop(0, cpw // NBUF, pipe_body, 0)
        plsc.subcore_barrier()

        pltpu.sync_copy(acc.at[pl.ds(r0, rows_per_tile)],
                        out_hbm.at[pl.ds(cid * n_pad + r0, rows_per_tile)])

    return scatter_kernel(x, src_p, dst_p), n_pad


def _combine(x, p0, p1, n, d, scale):
    block = 2000

    def body(x_ref, a_ref, b_ref, o_ref):
        o_ref[...] = jnp.maximum(
            (x_ref[...] + a_ref[...] + b_ref[...]) * scale, 0.0)

    spec = pl.BlockSpec((block, d), lambda i: (i, 0))
    return pl.pallas_call(
        body,
        grid=(n // block,),
        in_specs=[spec, spec, spec],
        out_specs=spec,
        out_shape=jax.ShapeDtypeStruct((n, d), jnp.float32),
    )(x, p0, p1)


def kernel(x, edge_index, edge_weights, W_w, b_w, att):
    n, d = x.shape
    e = edge_index.shape[1]
    out_dim = att.shape[1]

    cpw0, cpw1 = _split_chunks(e)
    cpw_max = max(cpw0, cpw1)
    ep = (cpw0 + cpw1) * NS * CHUNK
    pad = ep - e

    def _per_worker(flat, fill):
        flat = jnp.concatenate(
            [flat, jnp.full((pad,), fill, jnp.int32)])
        lt0 = NS * cpw0 * CHUNK
        p0 = jnp.pad(flat[:lt0].reshape(NS, cpw0 * CHUNK),
                     ((0, 0), (0, (cpw_max - cpw0) * CHUNK)),
                     constant_values=fill)
        p1 = jnp.pad(flat[lt0:].reshape(NS, cpw1 * CHUNK),
                     ((0, 0), (0, (cpw_max - cpw1) * CHUNK)),
                     constant_values=fill)
        return jnp.concatenate([p0, p1]).reshape(NC * NS, cpw_max, CHUNK)

    src_p = _per_worker(edge_index[0], 0)
    dst_p = _per_worker(edge_index[1], n)

    partials, n_pad = _sc_scatter_partials(x, src_p, dst_p, n, d, cpw0, cpw1)
    return _combine(x, partials[:n], partials[n_pad:n_pad + n], n, d,
                    1.0 / out_dim)
